# Initial kernel scaffold; baseline (speedup 1.0000x reference)
import jax
import jax.numpy as jnp
from jax.experimental import pallas as pl


def kernel(user_id, item_id, cate_id, hist_item_ids, W_user_id, W_item_id, W_cate_id, W_hist_item_ids):
    raise NotImplementedError("write your pallas kernel here")


# speedup vs baseline: 1.1172x; 1.1172x over previous
import functools

import jax
import jax.numpy as jnp
from jax import lax
from jax.experimental import pallas as pl
from jax.experimental.pallas import tpu as pltpu
from jax.experimental.pallas import tpu_sc as plsc

B = 16384
D = 32
L = 50
NC, NS, NL = 2, 16, 16
NW = NC * NS
BPW = B // NW
CH = 16
NCH = BPW // CH
HI = CH * L


def _sc_embed(user_id, item_id, cate_id, hist_flat, W_user, W_item, W_cate, W_hist):
    mesh = plsc.VectorSubcoreMesh(core_axis_name="c", subcore_axis_name="s")

    @functools.partial(
        pl.kernel,
        mesh=mesh,
        compiler_params=pltpu.CompilerParams(
            needs_layout_passes=False, use_tc_tiling_on_sc=False),
        out_type=[
            jax.ShapeDtypeStruct((B, D), jnp.float32),
            jax.ShapeDtypeStruct((B, D), jnp.float32),
            jax.ShapeDtypeStruct((B, D), jnp.float32),
            jax.ShapeDtypeStruct((B, D), jnp.float32),
        ],
        scratch_types=[
            pltpu.VMEM((BPW,), jnp.int32),
            pltpu.VMEM((BPW, D), jnp.float32),
            pltpu.VMEM((HI,), jnp.int32),
            pltpu.VMEM((HI, D), jnp.float32),
            pltpu.VMEM((CH, D), jnp.float32),
            pltpu.SemaphoreType.DMA,
        ],
    )
    def k(uid_h, iid_h, cid_h, hist_h, Wu_h, Wi_h, Wc_h, Wh_h,
          ou_h, oi_h, oc_h, oh_h,
          idx_s, rows_s, hidx, hrows, ostage, sem):
        wid = lax.axis_index("s") * NC + lax.axis_index("c")
        base = wid * BPW

        for idx_hbm, W_hbm, out_hbm in (
                (uid_h, Wu_h, ou_h), (iid_h, Wi_h, oi_h), (cid_h, Wc_h, oc_h)):
            pltpu.sync_copy(idx_hbm.at[pl.ds(base, BPW)], idx_s)
            pltpu.async_copy(W_hbm.at[idx_s], rows_s, sem).wait()
            pltpu.sync_copy(rows_s, out_hbm.at[pl.ds(base, BPW)])

        lane = lax.iota(jnp.int32, NL)
        tail_ok = (lane + (L - NL)) >= 3 * NL

        def chunk(c, carry):
            off = (base + c * CH) * L
            pltpu.sync_copy(hist_h.at[pl.ds(off, HI)], hidx)
            pltpu.async_copy(Wh_h.at[hidx], hrows, sem).wait()
            for r in range(CH):
                b = r * L
                i0 = hidx[pl.ds(b, NL)]
                i1 = hidx[pl.ds(b + NL, NL)]
                i2 = hidx[pl.ds(b + 2 * NL, NL)]
                i3 = hidx[pl.ds(b + L - NL, NL)]
                n = (jnp.where(i0 != 0, 1.0, 0.0)
                     + jnp.where(i1 != 0, 1.0, 0.0)
                     + jnp.where(i2 != 0, 1.0, 0.0)
                     + jnp.where((i3 != 0) & tail_ok, 1.0, 0.0))
                s = jnp.sum(n)
                invv = 1.0 / (jnp.broadcast_to(s, (NL,)) + 1e-8)
                a0 = hrows[b, pl.ds(0, NL)]
                a1 = hrows[b, pl.ds(NL, NL)]
                for l in range(1, L):
                    a0 = a0 + hrows[b + l, pl.ds(0, NL)]
                    a1 = a1 + hrows[b + l, pl.ds(NL, NL)]
                ostage[r, pl.ds(0, NL)] = a0 * invv
                ostage[r, pl.ds(NL, NL)] = a1 * invv
            pltpu.sync_copy(ostage, oh_h.at[pl.ds(base + c * CH, CH)])
            return carry

        lax.fori_loop(0, NCH, chunk, 0)

    return k(user_id, item_id, cate_id, hist_flat, W_user, W_item, W_cate, W_hist)


def kernel(user_id, item_id, cate_id, hist_item_ids,
           W_user_id, W_item_id, W_cate_id, W_hist_item_ids):
    hist_flat = hist_item_ids.reshape(-1)
    out = _sc_embed(user_id, item_id, cate_id, hist_flat,
                    W_user_id, W_item_id, W_cate_id, W_hist_item_ids)
    return tuple(out)
